# baseline (speedup 1.0000x reference)
import functools

import jax
import jax.numpy as jnp
from jax import lax
from jax.experimental import pallas as pl
from jax.experimental.pallas import tpu as pltpu
from jax.experimental.pallas import tpu_sc as plsc

NC = 2
NS = 16
NW = NC * NS

G = 40
RPC = 4


@functools.partial(jax.jit, static_argnames=("b", "l", "d"))
def _emb_lookup(x, pos_enc, table, b, l, d):
    chunks_per_w = b // (RPC * NW)
    ng = RPC * l // G

    mesh = plsc.VectorSubcoreMesh(
        core_axis_name="c", subcore_axis_name="s", num_cores=NC, num_subcores=NS
    )

    @functools.partial(
        pl.kernel,
        out_type=jax.ShapeDtypeStruct((b, l, 2 * d), jnp.float32),
        mesh=mesh,
        scratch_types=[
            pltpu.VMEM((2, RPC, l), jnp.int32),
            pltpu.VMEM((2, RPC, l, d), jnp.float32),
            pltpu.VMEM((l, d), jnp.float32),
            pltpu.SemaphoreType.DMA,
            pltpu.SemaphoreType.DMA,
            pltpu.SemaphoreType.DMA,
            pltpu.SemaphoreType.DMA,
        ],
        compiler_params=pltpu.CompilerParams(use_tc_tiling_on_sc=False),
    )
    def k(x_hbm, pos_hbm, table_hbm, out_hbm, idx_v, rows_v, pos_v,
          gsem0, gsem1, ssem0, ssem1):
        gsem = (gsem0, gsem1)
        ssem = (ssem0, ssem1)
        wid = lax.axis_index("s") * NC + lax.axis_index("c")
        base_chunk = wid * chunks_per_w
        pltpu.sync_copy(pos_hbm.at[0, pl.ds(0, l)], pos_v)

        def gather_descs(buf):
            descs = []
            for j in range(ng):
                rep, half = divmod(j, l // G)
                sl = pl.ds(half * G, G)
                descs.append(
                    pltpu.make_async_copy(
                        table_hbm.at[idx_v.at[buf, rep, sl]],
                        rows_v.at[buf, rep, sl],
                        gsem[buf],
                    )
                )
            return descs

        def issue_chunk(buf, cid):
            pltpu.sync_copy(x_hbm.at[pl.ds(cid * RPC, RPC)], idx_v.at[buf])
            for cp in gather_descs(buf):
                cp.start()

        def scatter_desc(buf, cid):
            return pltpu.make_async_copy(
                rows_v.at[buf],
                out_hbm.at[pl.ds(cid * RPC, RPC), :, pl.ds(0, d)],
                ssem[buf],
            )

        issue_chunk(0, base_chunk)

        def pair_body(t, carry):
            for buf in range(2):
                c = 2 * t + buf
                cid = base_chunk + c
                for cp in gather_descs(buf):
                    cp.wait()
                nxt = 1 - buf

                @pl.when(c >= 1)
                def _():
                    scatter_desc(nxt, cid - 1).wait()

                @pl.when(c + 1 < chunks_per_w)
                def _():
                    issue_chunk(nxt, cid + 1)

                def add_body(rr, carry2):
                    for rep in range(RPC):
                        for kk in range(d // 16):
                            sl = pl.ds(kk * 16, 16)
                            rows_v[buf, rep, rr, sl] = (
                                rows_v[buf, rep, rr, sl] + pos_v[rr, sl]
                            )
                    return carry2

                lax.fori_loop(0, l, add_body, 0)
                scatter_desc(buf, cid).start()
            return carry

        lax.fori_loop(0, chunks_per_w // 2, pair_body, 0)
        scatter_desc(1, base_chunk + chunks_per_w - 1).wait()

    return k(x, pos_enc, table)


def kernel(x, table, pos_enc):
    b, l = x.shape
    _, d = table.shape
    assert b % (RPC * NW) == 0 and (b // (RPC * NW)) % 2 == 0
    assert l % G == 0 and d % 16 == 0 and pos_enc.shape[1] >= l
    v = table.shape[0]
    nshards = 2
    vs = v // nshards
    flat = jnp.concatenate(
        [
            lax.optimization_barrier(table[i * vs:(i + 1) * vs].reshape(-1))
            for i in range(nshards)
        ]
    )
    out = _emb_lookup(x, pos_enc, flat.reshape(v, d), b, l, d)
    return out[:, :, :d]


# speedup vs baseline: 2.4028x; 2.4028x over previous
import functools

import jax
import jax.numpy as jnp
from jax import lax
from jax.experimental import pallas as pl
from jax.experimental.pallas import tpu as pltpu
from jax.experimental.pallas import tpu_sc as plsc

NC = 2
NS = 16
NW = NC * NS

G = 40
RPC = 4


@functools.partial(jax.jit, static_argnames=("b", "l", "d"))
def _emb_lookup(x, pos_enc, table, b, l, d):
    chunks_per_w = b // (RPC * NW)
    ng = RPC * l // G

    mesh = plsc.VectorSubcoreMesh(
        core_axis_name="c", subcore_axis_name="s", num_cores=NC, num_subcores=NS
    )

    @functools.partial(
        pl.kernel,
        out_type=jax.ShapeDtypeStruct((b, l, 2 * d), jnp.float32),
        mesh=mesh,
        scratch_types=[
            pltpu.VMEM((2, RPC, l), jnp.int32),
            pltpu.VMEM((2, RPC, l, d), jnp.float32),
            pltpu.VMEM((l, d), jnp.float32),
            pltpu.SemaphoreType.DMA,
            pltpu.SemaphoreType.DMA,
            pltpu.SemaphoreType.DMA,
            pltpu.SemaphoreType.DMA,
        ],
        compiler_params=pltpu.CompilerParams(use_tc_tiling_on_sc=False),
    )
    def k(x_hbm, pos_hbm, table_hbm, out_hbm, idx_v, rows_v, pos_v,
          gsem0, gsem1, ssem0, ssem1):
        gsem = (gsem0, gsem1)
        ssem = (ssem0, ssem1)
        wid = lax.axis_index("s") * NC + lax.axis_index("c")
        base_chunk = wid * chunks_per_w
        pltpu.sync_copy(pos_hbm.at[0, pl.ds(0, l)], pos_v)

        def gather_descs(buf):
            descs = []
            for j in range(ng):
                rep, half = divmod(j, l // G)
                sl = pl.ds(half * G, G)
                descs.append(
                    pltpu.make_async_copy(
                        table_hbm.at[idx_v.at[buf, rep, sl]],
                        rows_v.at[buf, rep, sl],
                        gsem[buf],
                    )
                )
            return descs

        def issue_chunk(buf, cid):
            pltpu.sync_copy(x_hbm.at[pl.ds(cid * RPC, RPC)], idx_v.at[buf])
            for cp in gather_descs(buf):
                cp.start()

        def scatter_desc(buf, cid):
            return pltpu.make_async_copy(
                rows_v.at[buf],
                out_hbm.at[pl.ds(cid * RPC, RPC), :, pl.ds(0, d)],
                ssem[buf],
            )

        issue_chunk(0, base_chunk)

        def pair_body(t, carry):
            for buf in range(2):
                c = 2 * t + buf
                cid = base_chunk + c
                for cp in gather_descs(buf):
                    cp.wait()
                nxt = 1 - buf

                @pl.when(c >= 1)
                def _():
                    scatter_desc(nxt, cid - 1).wait()

                @pl.when(c + 1 < chunks_per_w)
                def _():
                    issue_chunk(nxt, cid + 1)

                def add_body(rr, carry2):
                    for rep in range(RPC):
                        for kk in range(d // 16):
                            sl = pl.ds(kk * 16, 16)
                            rows_v[buf, rep, rr, sl] = (
                                rows_v[buf, rep, rr, sl] + pos_v[rr, sl]
                            )
                    return carry2

                lax.fori_loop(0, l, add_body, 0)
                scatter_desc(buf, cid).start()
            return carry

        lax.fori_loop(0, chunks_per_w // 2, pair_body, 0)
        scatter_desc(1, base_chunk + chunks_per_w - 1).wait()

    return k(x, pos_enc, table)


def kernel(x, table, pos_enc):
    b, l = x.shape
    _, d = table.shape
    assert b % (RPC * NW) == 0 and (b // (RPC * NW)) % 2 == 0
    assert l % G == 0 and d % 16 == 0 and pos_enc.shape[1] >= l
    out = _emb_lookup(x, pos_enc, table, b, l, d)
    return out[:, :, :d]
